# baseline (speedup 1.0000x reference)
import functools

import jax
import jax.numpy as jnp
from jax import lax
from jax.experimental import pallas as pl
from jax.experimental.pallas import tpu as pltpu
from jax.experimental.pallas import tpu_sc as plsc

VOCAB_P1 = 100001
EMB = 128
BATCH = 16384
WINDOW = 5
N_CLASS = 50

NC = 2
NS = 16
NW = NC * NS

S = 2
SB = BATCH // S
B_PER_W = SB // NW
CHUNK = 128
NB = B_PER_W // CHUNK
NCHUNK = WINDOW * NB
K = 2
NGRP = NCHUNK // K


def _gather_body(idx_hbm, table_hbm, out_hbm, idx_v,
                 a0, a1, b0, b1, gsa, gsb, wsa, wsb):
  wid = lax.axis_index("s") * NC + lax.axis_index("c")
  base = wid * B_PER_W
  pltpu.sync_copy(idx_hbm.at[wid], idx_v)
  bufs = [(a0, a1, gsa, wsa), (b0, b1, gsb, wsb)]

  def fire_gathers(grp):
    r0, r1, gs, _ = bufs[grp % 2]
    j = grp * K
    return [pltpu.async_copy(table_hbm.at[idx_v.at[j]], r0, gs),
            pltpu.async_copy(table_hbm.at[idx_v.at[j + 1]], r1, gs)]

  def fire_writes(grp):
    r0, r1, _, ws = bufs[grp % 2]
    hs = []
    for k, r in ((0, r0), (1, r1)):
      j = grp * K + k
      w, c = j // NB, j % NB
      hs.append(pltpu.async_copy(
          r, out_hbm.at[w, pl.ds(base + c * CHUNK, CHUNK)], ws))
    return hs

  g_handles = {0: fire_gathers(0)}
  w_handles = {}
  for grp in range(NGRP):
    if grp + 1 < NGRP:
      if grp >= 1:
        for h in w_handles[grp - 1]:
          h.wait()
      g_handles[grp + 1] = fire_gathers(grp + 1)
    for h in g_handles[grp]:
      h.wait()
    w_handles[grp] = fire_writes(grp)
  for grp in (NGRP - 2, NGRP - 1):
    for h in w_handles[grp]:
      h.wait()


_sc_gather = functools.partial(
    pl.kernel,
    out_type=jax.ShapeDtypeStruct((WINDOW, SB, EMB), jnp.float32),
    mesh=plsc.VectorSubcoreMesh(core_axis_name="c", subcore_axis_name="s"),
    scratch_types=[
        pltpu.VMEM((NCHUNK, CHUNK), jnp.int32),
        pltpu.VMEM((CHUNK, EMB), jnp.float32),
        pltpu.VMEM((CHUNK, EMB), jnp.float32),
        pltpu.VMEM((CHUNK, EMB), jnp.float32),
        pltpu.VMEM((CHUNK, EMB), jnp.float32),
        pltpu.SemaphoreType.DMA,
        pltpu.SemaphoreType.DMA,
        pltpu.SemaphoreType.DMA,
        pltpu.SemaphoreType.DMA,
    ],
)(_gather_body)


BM = 512


def _matmul_body(g_ref, w_ref, b_ref, out_ref):
  acc = b_ref[...]
  for w in range(WINDOW):
    acc = acc + lax.dot_general(
        g_ref[w], w_ref[w],
        dimension_numbers=(((1,), (1,)), ((), ())),
        preferred_element_type=jnp.float32,
    )
  out_ref[...] = acc


def _matmul_body_acc(g_ref, w_ref, b_ref, prev_ref, out_ref):
  del prev_ref
  _matmul_body(g_ref, w_ref, b_ref, out_ref)


def _tc_matmul(g, wr, b2d, stage, prev=None):
  blocks = SB // BM
  out_map = lambda i: (stage * blocks + i, 0)
  if stage == 0:
    return pl.pallas_call(
        _matmul_body,
        grid=(blocks,),
        in_specs=[
            pl.BlockSpec((WINDOW, BM, EMB), lambda i: (0, i, 0)),
            pl.BlockSpec((WINDOW, N_CLASS, EMB), lambda i: (0, 0, 0)),
            pl.BlockSpec((1, N_CLASS), lambda i: (0, 0)),
        ],
        out_specs=pl.BlockSpec((BM, N_CLASS), out_map),
        out_shape=jax.ShapeDtypeStruct((BATCH, N_CLASS), jnp.float32),
    )(g, wr, b2d)
  return pl.pallas_call(
      _matmul_body_acc,
      grid=(blocks,),
      in_specs=[
          pl.BlockSpec((WINDOW, BM, EMB), lambda i: (0, i, 0)),
          pl.BlockSpec((WINDOW, N_CLASS, EMB), lambda i: (0, 0, 0)),
          pl.BlockSpec((1, N_CLASS), lambda i: (0, 0)),
          pl.BlockSpec(memory_space=pltpu.MemorySpace.HBM),
      ],
      out_specs=pl.BlockSpec((BM, N_CLASS), out_map),
      out_shape=jax.ShapeDtypeStruct((BATCH, N_CLASS), jnp.float32),
      input_output_aliases={3: 0},
  )(g, wr, b2d, prev)


@jax.jit
def kernel(x, table, W, b):
  idx = (x.astype(jnp.int32).T
         .reshape(WINDOW, S, NW, NB, CHUNK)
         .transpose(1, 2, 0, 3, 4)
         .reshape(S, NW, NCHUNK, CHUNK))
  wr = W.reshape(N_CLASS, WINDOW, EMB).transpose(1, 0, 2)
  b2d = b.reshape(1, N_CLASS)
  out = None
  for s in range(S):
    g = _sc_gather(idx[s], table)
    out = _tc_matmul(g, wr, b2d, s, out)
  return out


# speedup vs baseline: 1.0689x; 1.0689x over previous
import functools

import jax
import jax.numpy as jnp
from jax import lax
from jax.experimental import pallas as pl
from jax.experimental.pallas import tpu as pltpu
from jax.experimental.pallas import tpu_sc as plsc

VOCAB_P1 = 100001
EMB = 128
BATCH = 16384
WINDOW = 5
N_CLASS = 50

NC = 2
NS = 16
NW = NC * NS

S = 2
SB = BATCH // S
B_PER_W = SB // NW
CHUNK = 128
NB = B_PER_W // CHUNK
NCHUNK = WINDOW * NB
K = 2
NGRP = NCHUNK // K


def _gather_body(idx_hbm, table_hbm, out_hbm, idx_v,
                 a0, a1, b0, b1, gsa, gsb, wsa, wsb):
  wid = lax.axis_index("s") * NC + lax.axis_index("c")
  base = wid * B_PER_W
  pltpu.sync_copy(idx_hbm.at[wid], idx_v)
  bufs = [(a0, a1, gsa, wsa), (b0, b1, gsb, wsb)]

  def fire_gathers(grp):
    r0, r1, gs, _ = bufs[grp % 2]
    j = grp * K
    return [pltpu.async_copy(table_hbm.at[idx_v.at[j]], r0, gs),
            pltpu.async_copy(table_hbm.at[idx_v.at[j + 1]], r1, gs)]

  def fire_writes(grp):
    r0, r1, _, ws = bufs[grp % 2]
    hs = []
    for k, r in ((0, r0), (1, r1)):
      j = grp * K + k
      w, c = j // NB, j % NB
      hs.append(pltpu.async_copy(
          r, out_hbm.at[w, pl.ds(base + c * CHUNK, CHUNK)], ws))
    return hs

  g_handles = {0: fire_gathers(0)}
  w_handles = {}
  for grp in range(NGRP):
    if grp + 1 < NGRP:
      if grp >= 1:
        for h in w_handles[grp - 1]:
          h.wait()
      g_handles[grp + 1] = fire_gathers(grp + 1)
    for h in g_handles[grp]:
      h.wait()
    w_handles[grp] = fire_writes(grp)
  for grp in (NGRP - 2, NGRP - 1):
    for h in w_handles[grp]:
      h.wait()


_sc_gather = functools.partial(
    pl.kernel,
    out_type=jax.ShapeDtypeStruct((WINDOW, SB, EMB), jnp.float32),
    mesh=plsc.VectorSubcoreMesh(core_axis_name="c", subcore_axis_name="s"),
    scratch_types=[
        pltpu.VMEM((NCHUNK, CHUNK), jnp.int32),
        pltpu.VMEM((CHUNK, EMB), jnp.float32),
        pltpu.VMEM((CHUNK, EMB), jnp.float32),
        pltpu.VMEM((CHUNK, EMB), jnp.float32),
        pltpu.VMEM((CHUNK, EMB), jnp.float32),
        pltpu.SemaphoreType.DMA,
        pltpu.SemaphoreType.DMA,
        pltpu.SemaphoreType.DMA,
        pltpu.SemaphoreType.DMA,
    ],
)(_gather_body)


BM = 1024


def _matmul_body(g_ref, w_ref, b_ref, out_ref):
  acc = b_ref[...]
  for w in range(WINDOW):
    acc = acc + lax.dot_general(
        g_ref[w], w_ref[w],
        dimension_numbers=(((1,), (1,)), ((), ())),
        preferred_element_type=jnp.float32,
    )
  out_ref[...] = acc


def _matmul_body_acc(g_ref, w_ref, b_ref, prev_ref, out_ref):
  del prev_ref
  _matmul_body(g_ref, w_ref, b_ref, out_ref)


def _tc_matmul(g, wr, b2d, stage, prev=None):
  blocks = SB // BM
  out_map = lambda i: (stage * blocks + i, 0)
  if stage == 0:
    return pl.pallas_call(
        _matmul_body,
        grid=(blocks,),
        in_specs=[
            pl.BlockSpec((WINDOW, BM, EMB), lambda i: (0, i, 0)),
            pl.BlockSpec((WINDOW, N_CLASS, EMB), lambda i: (0, 0, 0)),
            pl.BlockSpec((1, N_CLASS), lambda i: (0, 0)),
        ],
        out_specs=pl.BlockSpec((BM, N_CLASS), out_map),
        out_shape=jax.ShapeDtypeStruct((BATCH, N_CLASS), jnp.float32),
    )(g, wr, b2d)
  return pl.pallas_call(
      _matmul_body_acc,
      grid=(blocks,),
      in_specs=[
          pl.BlockSpec((WINDOW, BM, EMB), lambda i: (0, i, 0)),
          pl.BlockSpec((WINDOW, N_CLASS, EMB), lambda i: (0, 0, 0)),
          pl.BlockSpec((1, N_CLASS), lambda i: (0, 0)),
          pl.BlockSpec(memory_space=pltpu.MemorySpace.HBM),
      ],
      out_specs=pl.BlockSpec((BM, N_CLASS), out_map),
      out_shape=jax.ShapeDtypeStruct((BATCH, N_CLASS), jnp.float32),
      input_output_aliases={3: 0},
  )(g, wr, b2d, prev)


@jax.jit
def kernel(x, table, W, b):
  idx = (x.astype(jnp.int32).T
         .reshape(WINDOW, S, NW, NB, CHUNK)
         .transpose(1, 2, 0, 3, 4)
         .reshape(S, NW, NCHUNK, CHUNK))
  wr = W.reshape(N_CLASS, WINDOW, EMB).transpose(1, 0, 2)
  b2d = b.reshape(1, N_CLASS)
  out = None
  for s in range(S):
    g = _sc_gather(idx[s], table)
    out = _tc_matmul(g, wr, b2d, s, out)
  return out


# speedup vs baseline: 1.1148x; 1.0429x over previous
import functools

import jax
import jax.numpy as jnp
from jax import lax
from jax.experimental import pallas as pl
from jax.experimental.pallas import tpu as pltpu
from jax.experimental.pallas import tpu_sc as plsc

VOCAB_P1 = 100001
EMB = 128
BATCH = 16384
WINDOW = 5
N_CLASS = 50

NC = 2
NS = 16
NW = NC * NS

S = 1
SB = BATCH // S
B_PER_W = SB // NW
CHUNK = 128
NB = B_PER_W // CHUNK
NCHUNK = WINDOW * NB
K = 2
NGRP = NCHUNK // K


def _gather_body(idx_hbm, table_hbm, out_hbm, idx_v,
                 a0, a1, b0, b1, gsa, gsb, wsa, wsb):
  wid = lax.axis_index("s") * NC + lax.axis_index("c")
  base = wid * B_PER_W
  pltpu.sync_copy(idx_hbm.at[wid], idx_v)
  bufs = [(a0, a1, gsa, wsa), (b0, b1, gsb, wsb)]

  def fire_gathers(grp):
    r0, r1, gs, _ = bufs[grp % 2]
    j = grp * K
    return [pltpu.async_copy(table_hbm.at[idx_v.at[j]], r0, gs),
            pltpu.async_copy(table_hbm.at[idx_v.at[j + 1]], r1, gs)]

  def fire_writes(grp):
    r0, r1, _, ws = bufs[grp % 2]
    hs = []
    for k, r in ((0, r0), (1, r1)):
      j = grp * K + k
      w, c = j // NB, j % NB
      hs.append(pltpu.async_copy(
          r, out_hbm.at[w, pl.ds(base + c * CHUNK, CHUNK)], ws))
    return hs

  g_handles = {0: fire_gathers(0)}
  w_handles = {}
  for grp in range(NGRP):
    if grp + 1 < NGRP:
      if grp >= 1:
        for h in w_handles[grp - 1]:
          h.wait()
      g_handles[grp + 1] = fire_gathers(grp + 1)
    for h in g_handles[grp]:
      h.wait()
    w_handles[grp] = fire_writes(grp)
  for grp in (NGRP - 2, NGRP - 1):
    for h in w_handles[grp]:
      h.wait()


_sc_gather = functools.partial(
    pl.kernel,
    out_type=jax.ShapeDtypeStruct((WINDOW, SB, EMB), jnp.float32),
    mesh=plsc.VectorSubcoreMesh(core_axis_name="c", subcore_axis_name="s"),
    scratch_types=[
        pltpu.VMEM((NCHUNK, CHUNK), jnp.int32),
        pltpu.VMEM((CHUNK, EMB), jnp.float32),
        pltpu.VMEM((CHUNK, EMB), jnp.float32),
        pltpu.VMEM((CHUNK, EMB), jnp.float32),
        pltpu.VMEM((CHUNK, EMB), jnp.float32),
        pltpu.SemaphoreType.DMA,
        pltpu.SemaphoreType.DMA,
        pltpu.SemaphoreType.DMA,
        pltpu.SemaphoreType.DMA,
    ],
)(_gather_body)


BM = 2048


def _matmul_body(g_ref, w_ref, b_ref, out_ref):
  acc = b_ref[...]
  for w in range(WINDOW):
    acc = acc + lax.dot_general(
        g_ref[w], w_ref[w],
        dimension_numbers=(((1,), (1,)), ((), ())),
        preferred_element_type=jnp.float32,
    )
  out_ref[...] = acc


def _matmul_body_acc(g_ref, w_ref, b_ref, prev_ref, out_ref):
  del prev_ref
  _matmul_body(g_ref, w_ref, b_ref, out_ref)


def _tc_matmul(g, wr, b2d, stage, prev=None):
  blocks = SB // BM
  out_map = lambda i: (stage * blocks + i, 0)
  if stage == 0:
    return pl.pallas_call(
        _matmul_body,
        grid=(blocks,),
        in_specs=[
            pl.BlockSpec((WINDOW, BM, EMB), lambda i: (0, i, 0)),
            pl.BlockSpec((WINDOW, N_CLASS, EMB), lambda i: (0, 0, 0)),
            pl.BlockSpec((1, N_CLASS), lambda i: (0, 0)),
        ],
        out_specs=pl.BlockSpec((BM, N_CLASS), out_map),
        out_shape=jax.ShapeDtypeStruct((BATCH, N_CLASS), jnp.float32),
    )(g, wr, b2d)
  return pl.pallas_call(
      _matmul_body_acc,
      grid=(blocks,),
      in_specs=[
          pl.BlockSpec((WINDOW, BM, EMB), lambda i: (0, i, 0)),
          pl.BlockSpec((WINDOW, N_CLASS, EMB), lambda i: (0, 0, 0)),
          pl.BlockSpec((1, N_CLASS), lambda i: (0, 0)),
          pl.BlockSpec(memory_space=pltpu.MemorySpace.HBM),
      ],
      out_specs=pl.BlockSpec((BM, N_CLASS), out_map),
      out_shape=jax.ShapeDtypeStruct((BATCH, N_CLASS), jnp.float32),
      input_output_aliases={3: 0},
  )(g, wr, b2d, prev)


@jax.jit
def kernel(x, table, W, b):
  idx = (x.astype(jnp.int32).T
         .reshape(WINDOW, S, NW, NB, CHUNK)
         .transpose(1, 2, 0, 3, 4)
         .reshape(S, NW, NCHUNK, CHUNK))
  wr = W.reshape(N_CLASS, WINDOW, EMB).transpose(1, 0, 2)
  b2d = b.reshape(1, N_CLASS)
  out = None
  for s in range(S):
    g = _sc_gather(idx[s], table)
    out = _tc_matmul(g, wr, b2d, s, out)
  return out


# speedup vs baseline: 1.1251x; 1.0093x over previous
import functools

import jax
import jax.numpy as jnp
from jax import lax
from jax.experimental import pallas as pl
from jax.experimental.pallas import tpu as pltpu
from jax.experimental.pallas import tpu_sc as plsc

VOCAB_P1 = 100001
EMB = 128
BATCH = 16384
WINDOW = 5
N_CLASS = 50

NC = 2
NS = 16
NW = NC * NS

S = 1
SB = BATCH // S
B_PER_W = SB // NW
CHUNK = 128
NB = B_PER_W // CHUNK
NCHUNK = WINDOW * NB
K = 2
NGRP = NCHUNK // K


NBUF = 3


def _gather_body(idx_hbm, table_hbm, out_hbm, idx_v, *bufs_and_sems):
  wid = lax.axis_index("s") * NC + lax.axis_index("c")
  base = wid * B_PER_W
  pltpu.sync_copy(idx_hbm.at[wid], idx_v)
  rows = bufs_and_sems[:NBUF * K]
  gsems = bufs_and_sems[NBUF * K:NBUF * K + NBUF]
  wsems = bufs_and_sems[NBUF * K + NBUF:]
  bufs = [(rows[g * K:(g + 1) * K], gsems[g], wsems[g]) for g in range(NBUF)]

  def fire_gathers(grp):
    rs, gs, _ = bufs[grp % NBUF]
    return [pltpu.async_copy(table_hbm.at[idx_v.at[grp * K + k]], rs[k], gs)
            for k in range(K)]

  def fire_writes(grp):
    rs, _, ws = bufs[grp % NBUF]
    hs = []
    for k in range(K):
      j = grp * K + k
      w, c = j // NB, j % NB
      hs.append(pltpu.async_copy(
          rs[k], out_hbm.at[w, pl.ds(base + c * CHUNK, CHUNK)], ws))
    return hs

  g_handles = {g: fire_gathers(g) for g in range(min(NBUF - 1, NGRP))}
  w_handles = {}
  for grp in range(NGRP):
    nxt = grp + NBUF - 1
    if nxt < NGRP:
      if grp >= 1:
        for h in w_handles[grp - 1]:
          h.wait()
      g_handles[nxt] = fire_gathers(nxt)
    for h in g_handles[grp]:
      h.wait()
    w_handles[grp] = fire_writes(grp)
  for grp in range(max(0, NGRP - NBUF), NGRP):
    if grp in w_handles:
      for h in w_handles[grp]:
        h.wait()


_sc_gather = functools.partial(
    pl.kernel,
    out_type=jax.ShapeDtypeStruct((WINDOW, SB, EMB), jnp.float32),
    mesh=plsc.VectorSubcoreMesh(core_axis_name="c", subcore_axis_name="s"),
    scratch_types=(
        [pltpu.VMEM((NCHUNK, CHUNK), jnp.int32)]
        + [pltpu.VMEM((CHUNK, EMB), jnp.float32) for _ in range(NBUF * K)]
        + [pltpu.SemaphoreType.DMA for _ in range(2 * NBUF)]
    ),
)(_gather_body)


BM = 2048


def _matmul_body(g_ref, w_ref, b_ref, out_ref):
  acc = b_ref[...]
  for w in range(WINDOW):
    acc = acc + lax.dot_general(
        g_ref[w], w_ref[w],
        dimension_numbers=(((1,), (1,)), ((), ())),
        preferred_element_type=jnp.float32,
    )
  out_ref[...] = acc


def _matmul_body_acc(g_ref, w_ref, b_ref, prev_ref, out_ref):
  del prev_ref
  _matmul_body(g_ref, w_ref, b_ref, out_ref)


def _tc_matmul(g, wr, b2d, stage, prev=None):
  blocks = SB // BM
  out_map = lambda i: (stage * blocks + i, 0)
  if stage == 0:
    return pl.pallas_call(
        _matmul_body,
        grid=(blocks,),
        in_specs=[
            pl.BlockSpec((WINDOW, BM, EMB), lambda i: (0, i, 0)),
            pl.BlockSpec((WINDOW, N_CLASS, EMB), lambda i: (0, 0, 0)),
            pl.BlockSpec((1, N_CLASS), lambda i: (0, 0)),
        ],
        out_specs=pl.BlockSpec((BM, N_CLASS), out_map),
        out_shape=jax.ShapeDtypeStruct((BATCH, N_CLASS), jnp.float32),
    )(g, wr, b2d)
  return pl.pallas_call(
      _matmul_body_acc,
      grid=(blocks,),
      in_specs=[
          pl.BlockSpec((WINDOW, BM, EMB), lambda i: (0, i, 0)),
          pl.BlockSpec((WINDOW, N_CLASS, EMB), lambda i: (0, 0, 0)),
          pl.BlockSpec((1, N_CLASS), lambda i: (0, 0)),
          pl.BlockSpec(memory_space=pltpu.MemorySpace.HBM),
      ],
      out_specs=pl.BlockSpec((BM, N_CLASS), out_map),
      out_shape=jax.ShapeDtypeStruct((BATCH, N_CLASS), jnp.float32),
      input_output_aliases={3: 0},
  )(g, wr, b2d, prev)


@jax.jit
def kernel(x, table, W, b):
  idx = (x.astype(jnp.int32).T
         .reshape(WINDOW, S, NW, NB, CHUNK)
         .transpose(1, 2, 0, 3, 4)
         .reshape(S, NW, NCHUNK, CHUNK))
  wr = W.reshape(N_CLASS, WINDOW, EMB).transpose(1, 0, 2)
  b2d = b.reshape(1, N_CLASS)
  out = None
  for s in range(S):
    g = _sc_gather(idx[s], table)
    out = _tc_matmul(g, wr, b2d, s, out)
  return out
